# baseline (speedup 1.0000x reference)
import functools

import jax
import jax.numpy as jnp
from jax import lax
from jax.experimental import pallas as pl
from jax.experimental.pallas import tpu as pltpu
from jax.experimental.pallas import tpu_sc as plsc

V = 6890
E = 110240
B = 8
C = 128

VP = 6912
RPT = VP // 16
EPT = 6912
EP = 16 * EPT
K = 128
NCHUNK = EPT // K
BPC = B // 2


def _make_spmm(dpad):
    mesh = plsc.VectorSubcoreMesh(core_axis_name="c", subcore_axis_name="s")

    @functools.partial(
        pl.kernel,
        out_type=jax.ShapeDtypeStruct((B * VP, dpad), jnp.float32),
        mesh=mesh,
        compiler_params=pltpu.CompilerParams(use_tc_tiling_on_sc=False),
        scratch_types=[
            pltpu.VMEM((NCHUNK, K), jnp.int32),
            pltpu.VMEM((NCHUNK, K), jnp.int32),
            pltpu.VMEM((NCHUNK, K), jnp.int32),
            pltpu.VMEM((NCHUNK, K), jnp.float32),
            pltpu.VMEM((3, K, dpad), jnp.float32),
            pltpu.VMEM((3, K, dpad), jnp.float32),
            pltpu.VMEM((RPT // 2, dpad), jnp.float32),
            pltpu.VMEM_SHARED((VP, dpad), jnp.float32),
            pltpu.SemaphoreType.DMA,
            pltpu.SemaphoreType.DMA,
            pltpu.SemaphoreType.DMA,
            pltpu.SemaphoreType.DMA,
        ],
    )
    def spmm(support_hbm, src_hbm, dst_hbm, ew_hbm, out_hbm,
             src_r, src_a, dst_r, ew_r, rows_a, rows_b, zero_v, acc_sh,
             sem_a, sem_b, sem_sa, sem_sb):
        c = lax.axis_index("c")
        s = lax.axis_index("s")

        pltpu.sync_copy(src_hbm.at[s], src_r)
        pltpu.sync_copy(dst_hbm.at[s], dst_r)
        pltpu.sync_copy(ew_hbm.at[s], ew_r)

        zvec = jnp.zeros((16,), jnp.float32)

        def _zfill(i, carry):
            for j in range(dpad // 16):
                zero_v[i, pl.ds(j * 16, 16)] = zvec
            return carry
        lax.fori_loop(0, RPT // 2, _zfill, 0)

        def _scale3(base, buf):
            def _q(q, cc):
                n = base + q
                for g in range(K // 16):
                    wv = ew_r[n, pl.ds(g * 16, 16)]
                    for l in range(16):
                        w = wv[l]
                        e = g * 16 + l
                        for j in range(dpad // 16):
                            buf[q, e, pl.ds(j * 16, 16)] = (
                                buf[q, e, pl.ds(j * 16, 16)] * w)
                return cc
            lax.fori_loop(0, 3, _q, 0)

        NSUPER = NCHUNK // 6

        def _batch(bi, carry):
            b = c * BPC + bi
            roff = b * VP

            def _adj(n, cc):
                for j in range(K // 16):
                    src_a[n, pl.ds(j * 16, 16)] = (
                        src_r[n, pl.ds(j * 16, 16)] + roff)
                return cc
            lax.fori_loop(0, NCHUNK, _adj, 0)

            pltpu.sync_copy(zero_v, acc_sh.at[pl.ds(s * RPT, RPT // 2)])
            pltpu.sync_copy(zero_v,
                            acc_sh.at[pl.ds(s * RPT + RPT // 2, RPT // 2)])
            plsc.subcore_barrier()

            for q in range(3):
                pltpu.async_copy(
                    support_hbm.at[src_a.at[q]], rows_a.at[q], sem_a)
                pltpu.async_copy(
                    support_hbm.at[src_a.at[3 + q]], rows_b.at[q], sem_b)

            def _super(t, cc):
                base = 6 * t

                for q in range(3):
                    pltpu.make_async_copy(
                        support_hbm.at[src_a.at[base + q]],
                        rows_a.at[q], sem_a).wait()
                _scale3(base, rows_a)
                for q in range(3):
                    pltpu.async_copy(
                        rows_a.at[q], acc_sh.at[dst_r.at[base + q]], sem_sa,
                        add=True)

                for q in range(3):
                    pltpu.make_async_copy(
                        support_hbm.at[src_a.at[base + 3 + q]],
                        rows_b.at[q], sem_b).wait()
                _scale3(base + 3, rows_b)
                for q in range(3):
                    pltpu.async_copy(
                        rows_b.at[q], acc_sh.at[dst_r.at[base + 3 + q]], sem_sb,
                        add=True)

                for q in range(3):
                    pltpu.make_async_copy(
                        rows_a.at[q], acc_sh.at[dst_r.at[base + q]],
                        sem_sa).wait()

                @pl.when(t < NSUPER - 1)
                def _():
                    for q in range(3):
                        pltpu.async_copy(
                            support_hbm.at[src_a.at[base + 6 + q]],
                            rows_a.at[q], sem_a)

                for q in range(3):
                    pltpu.make_async_copy(
                        rows_b.at[q], acc_sh.at[dst_r.at[base + 3 + q]],
                        sem_sb).wait()

                @pl.when(t < NSUPER - 1)
                def _():
                    for q in range(3):
                        pltpu.async_copy(
                            support_hbm.at[src_a.at[base + 9 + q]],
                            rows_b.at[q], sem_b)
                return cc

            lax.fori_loop(0, NSUPER, _super, 0)
            plsc.subcore_barrier()

            pltpu.sync_copy(acc_sh.at[pl.ds(s * RPT, RPT)],
                            out_hbm.at[pl.ds(roff + s * RPT, RPT)])
            plsc.subcore_barrier()
            return carry

        lax.fori_loop(0, BPC, _batch, 0)

    return spmm


_spmm64 = _make_spmm(64)
_spmm32 = _make_spmm(32)
_spmm16 = _make_spmm(16)


ROWS = B * VP
VB = 768
NVB = VP // VB

_DN0 = (((0,), (0,)), ((), ()))
_DN01 = (((0,), (1,)), ((), ()))


def _tc_pre_body(x_ref, f_ref, v_ref, w1_ref, b1_ref, wg_ref,
                 hvt_ref, s_ref):
    hvt = jnp.concatenate([x_ref[0], f_ref[0], v_ref[0]], axis=0)
    hvt_ref[0] = hvt
    t = lax.dot_general(jnp.maximum(hvt, 0.0), w1_ref[...], _DN0,
                        preferred_element_type=jnp.float32) + b1_ref[...]
    s_ref[0] = jnp.dot(t, wg_ref[...], preferred_element_type=jnp.float32)


def _tc_mid_body(hvt_ref, agg_ref, bg_ref, w2_ref, b2_ref,
                 w1_ref, b1_ref, wg_ref, hvo_ref, s_ref):
    u = jnp.maximum(agg_ref[0] + bg_ref[...], 0.0)
    ut = lax.dot_general(w2_ref[...], u, _DN01,
                         preferred_element_type=jnp.float32) + b2_ref[...]
    hvt = hvt_ref[0] + ut
    hvo_ref[0] = hvt
    t = lax.dot_general(jnp.maximum(hvt, 0.0), w1_ref[...], _DN0,
                        preferred_element_type=jnp.float32) + b1_ref[...]
    s_ref[0] = jnp.dot(t, wg_ref[...], preferred_element_type=jnp.float32)


def _tc_last_body(hvt_ref, agg_ref, bg_ref, w2_ref, b2_ref, ws_ref, wo_ref,
                  hvo_ref, s_ref):
    u = jnp.maximum(agg_ref[0] + bg_ref[...], 0.0)
    ut = lax.dot_general(w2_ref[...], u, _DN01,
                         preferred_element_type=jnp.float32) + b2_ref[...]
    skip = lax.dot_general(ws_ref[...], hvt_ref[0], _DN0,
                           preferred_element_type=jnp.float32)
    hvt = skip + ut
    hvo_ref[0] = hvt
    s_ref[0] = lax.dot_general(hvt, wo_ref[...], _DN0,
                               preferred_element_type=jnp.float32)


def _blk_spec(c, w):
    return pl.BlockSpec((1, c, w), lambda b, i: (b, 0, i))


def _full_spec(shape):
    return pl.BlockSpec(shape, lambda b, i: tuple(0 for _ in shape))


def _tc_pre(xp, fp, vp, w1, b1, wg):
    h = w1.shape[1]
    return pl.pallas_call(
        _tc_pre_body,
        grid=(B, NVB),
        in_specs=[_blk_spec(61, VB), _blk_spec(64, VB), _blk_spec(3, VB),
                  _full_spec(w1.shape), _full_spec(b1.shape),
                  _full_spec(wg.shape)],
        out_specs=[_blk_spec(C, VB), _blk_spec(VB, h)],
        out_shape=[jax.ShapeDtypeStruct((B, C, VP), jnp.float32),
                   jax.ShapeDtypeStruct((B, VP, h), jnp.float32)],
    )(xp, fp, vp, w1, b1, wg)


def _tc_mid(hvt, agg, bg, w2, b2, w1, b1, wg):
    h = w1.shape[1]
    ha = agg.shape[2]
    return pl.pallas_call(
        _tc_mid_body,
        grid=(B, NVB),
        in_specs=[_blk_spec(C, VB), _blk_spec(VB, ha), _full_spec(bg.shape),
                  _full_spec(w2.shape), _full_spec(b2.shape),
                  _full_spec(w1.shape), _full_spec(b1.shape),
                  _full_spec(wg.shape)],
        out_specs=[_blk_spec(C, VB), _blk_spec(VB, h)],
        out_shape=[jax.ShapeDtypeStruct((B, C, VP), jnp.float32),
                   jax.ShapeDtypeStruct((B, VP, h), jnp.float32)],
    )(hvt, agg, bg, w2, b2, w1, b1, wg)


def _tc_last(hvt, agg, bg, w2, b2, ws, wo):
    co = ws.shape[1]
    so = wo.shape[1]
    ha = agg.shape[2]
    return pl.pallas_call(
        _tc_last_body,
        grid=(B, NVB),
        in_specs=[_blk_spec(C, VB), _blk_spec(VB, ha), _full_spec(bg.shape),
                  _full_spec(w2.shape), _full_spec(b2.shape),
                  _full_spec(ws.shape), _full_spec(wo.shape)],
        out_specs=[_blk_spec(co, VB), _blk_spec(VB, so)],
        out_shape=[jax.ShapeDtypeStruct((B, co, VP), jnp.float32),
                   jax.ShapeDtypeStruct((B, VP, so), jnp.float32)],
    )(hvt, agg, bg, w2, b2, ws, wo)


def kernel(feature_input, x, vertices, edge_index, edge_weight,
           W1s, b1s, Wgs, bgs, W2s, b2s,
           W1f, b1f, Wgf, bgf, W2f, b2f, Wsf, Wo, bo):
    src = edge_index[0]
    dst = edge_index[1]
    epad = EP - E
    srcp = jnp.concatenate([src, jnp.zeros((epad,), jnp.int32)])
    dstp = jnp.concatenate([dst, jnp.zeros((epad,), jnp.int32)])
    ewp = jnp.concatenate([edge_weight, jnp.zeros((epad,), jnp.float32)])
    srcp = srcp.reshape(16, NCHUNK, K)
    dstp = dstp.reshape(16, NCHUNK, K)
    ewp = ewp.reshape(16, NCHUNK, K)

    vpad = ((0, 0), (0, 0), (0, VP - V))
    xp = jnp.pad(x, vpad)
    fp = jnp.pad(feature_input, vpad)
    vp = jnp.pad(vertices, vpad)

    hvt, s = _tc_pre(xp, fp, vp, W1s[0], b1s[0].reshape(1, -1), Wgs[0])
    for i in range(4):
        agg = _spmm64(s.reshape(ROWS, 64), srcp, dstp, ewp).reshape(B, VP, 64)
        if i < 3:
            hvt, s = _tc_mid(hvt, agg, bgs[i].reshape(1, -1),
                             W2s[i], b2s[i].reshape(-1, 1),
                             W1s[i + 1], b1s[i + 1].reshape(1, -1), Wgs[i + 1])
        else:
            hvt, s = _tc_mid(hvt, agg, bgs[3].reshape(1, -1),
                             W2s[3], b2s[3].reshape(-1, 1),
                             W1f, b1f.reshape(1, -1), Wgf)
    agg4 = _spmm32(s.reshape(ROWS, 32), srcp, dstp, ewp).reshape(B, VP, 32)
    wo_pad = jnp.pad(Wo, ((0, 0), (0, 16 - Wo.shape[1])))
    hvt5, s5 = _tc_last(hvt, agg4, bgf.reshape(1, -1), W2f,
                        b2f.reshape(-1, 1), Wsf, wo_pad)
    agg5 = _spmm16(s5.reshape(ROWS, 16), srcp, dstp, ewp)

    x_out = hvt5[:, :, :V]
    dv_v = agg5.reshape(B, VP, 16)[:, :V, :3] + bo
    dv = jnp.transpose(dv_v, (0, 2, 1))
    vertices_out = vertices + dv
    return (x_out, vertices_out, dv)


# speedup vs baseline: 1.0101x; 1.0101x over previous
import functools

import jax
import jax.numpy as jnp
from jax import lax
from jax.experimental import pallas as pl
from jax.experimental.pallas import tpu as pltpu
from jax.experimental.pallas import tpu_sc as plsc

V = 6890
E = 110240
B = 8
C = 128

VP = 6912
RPT = VP // 16
EPT = 6912
EP = 16 * EPT
K = 128
NCHUNK = EPT // K
BPC = B // 2


def _make_spmm(dpad):
    mesh = plsc.VectorSubcoreMesh(core_axis_name="c", subcore_axis_name="s")

    @functools.partial(
        pl.kernel,
        out_type=jax.ShapeDtypeStruct((B * VP, dpad), jnp.float32),
        mesh=mesh,
        compiler_params=pltpu.CompilerParams(use_tc_tiling_on_sc=False),
        scratch_types=[
            pltpu.VMEM((NCHUNK, K), jnp.int32),
            pltpu.VMEM((NCHUNK, K), jnp.int32),
            pltpu.VMEM((NCHUNK, K), jnp.int32),
            pltpu.VMEM((NCHUNK, K), jnp.float32),
            pltpu.VMEM((3, K, dpad), jnp.float32),
            pltpu.VMEM((3, K, dpad), jnp.float32),
            pltpu.VMEM((RPT // 2, dpad), jnp.float32),
            pltpu.VMEM_SHARED((VP, dpad), jnp.float32),
            pltpu.SemaphoreType.DMA,
            pltpu.SemaphoreType.DMA,
            pltpu.SemaphoreType.DMA,
            pltpu.SemaphoreType.DMA,
        ],
    )
    def spmm(support_hbm, src_hbm, dst_hbm, ew_hbm, out_hbm,
             src_r, src_a, dst_r, ew_r, rows_a, rows_b, zero_v, acc_sh,
             sem_a, sem_b, sem_sa, sem_sb):
        c = lax.axis_index("c")
        s = lax.axis_index("s")

        pltpu.sync_copy(src_hbm.at[s], src_r)
        pltpu.sync_copy(dst_hbm.at[s], dst_r)
        pltpu.sync_copy(ew_hbm.at[s], ew_r)

        zvec = jnp.zeros((16,), jnp.float32)

        def _zfill(i, carry):
            for j in range(dpad // 16):
                zero_v[i, pl.ds(j * 16, 16)] = zvec
            return carry
        lax.fori_loop(0, RPT // 2, _zfill, 0)

        def _scale3(base, buf):
            def _q(q, cc):
                n = base + q
                for g in range(K // 16):
                    wv = ew_r[n, pl.ds(g * 16, 16)]
                    for l in range(16):
                        w = wv[l]
                        e = g * 16 + l
                        for j in range(dpad // 16):
                            buf[q, e, pl.ds(j * 16, 16)] = (
                                buf[q, e, pl.ds(j * 16, 16)] * w)
                return cc
            lax.fori_loop(0, 3, _q, 0)

        NSUPER = NCHUNK // 6

        def _batch(bi, carry):
            b = c * BPC + bi
            roff = b * VP

            def _adj(n, cc):
                for j in range(K // 16):
                    src_a[n, pl.ds(j * 16, 16)] = (
                        src_r[n, pl.ds(j * 16, 16)] + roff)
                return cc
            lax.fori_loop(0, NCHUNK, _adj, 0)

            pltpu.sync_copy(zero_v, acc_sh.at[pl.ds(s * RPT, RPT // 2)])
            pltpu.sync_copy(zero_v,
                            acc_sh.at[pl.ds(s * RPT + RPT // 2, RPT // 2)])
            plsc.subcore_barrier()

            for q in range(3):
                pltpu.async_copy(
                    support_hbm.at[src_a.at[q]], rows_a.at[q], sem_a)
                pltpu.async_copy(
                    support_hbm.at[src_a.at[3 + q]], rows_b.at[q], sem_b)

            def _super(t, cc):
                base = 6 * t

                for q in range(3):
                    pltpu.make_async_copy(
                        support_hbm.at[src_a.at[base + q]],
                        rows_a.at[q], sem_a).wait()
                _scale3(base, rows_a)
                for q in range(3):
                    pltpu.async_copy(
                        rows_a.at[q], acc_sh.at[dst_r.at[base + q]], sem_sa,
                        add=True)

                for q in range(3):
                    pltpu.make_async_copy(
                        support_hbm.at[src_a.at[base + 3 + q]],
                        rows_b.at[q], sem_b).wait()
                _scale3(base + 3, rows_b)
                for q in range(3):
                    pltpu.async_copy(
                        rows_b.at[q], acc_sh.at[dst_r.at[base + 3 + q]], sem_sb,
                        add=True)

                for q in range(3):
                    pltpu.make_async_copy(
                        rows_a.at[q], acc_sh.at[dst_r.at[base + q]],
                        sem_sa).wait()

                @pl.when(t < NSUPER - 1)
                def _():
                    for q in range(3):
                        pltpu.async_copy(
                            support_hbm.at[src_a.at[base + 6 + q]],
                            rows_a.at[q], sem_a)

                for q in range(3):
                    pltpu.make_async_copy(
                        rows_b.at[q], acc_sh.at[dst_r.at[base + 3 + q]],
                        sem_sb).wait()

                @pl.when(t < NSUPER - 1)
                def _():
                    for q in range(3):
                        pltpu.async_copy(
                            support_hbm.at[src_a.at[base + 9 + q]],
                            rows_b.at[q], sem_b)
                return cc

            lax.fori_loop(0, NSUPER, _super, 0)
            plsc.subcore_barrier()

            pltpu.sync_copy(acc_sh.at[pl.ds(s * RPT, RPT)],
                            out_hbm.at[pl.ds(roff + s * RPT, RPT)])
            plsc.subcore_barrier()
            return carry

        lax.fori_loop(0, BPC, _batch, 0)

    return spmm


_spmm64 = _make_spmm(64)
_spmm32 = _make_spmm(32)
_spmm16 = _make_spmm(16)


ROWS = B * VP
VB = 768
NVB = VP // VB

_DN0 = (((0,), (0,)), ((), ()))
_DN01 = (((0,), (1,)), ((), ()))


def _tc_pre_body(x_ref, f_ref, v_ref, w1_ref, b1_ref, wg_ref,
                 hvt_ref, s_ref):
    hvt = jnp.concatenate([x_ref[0], f_ref[0], v_ref[0]], axis=0)
    hvt_ref[0] = hvt
    t = lax.dot_general(jnp.maximum(hvt, 0.0), w1_ref[...], _DN0,
                        preferred_element_type=jnp.float32) + b1_ref[...]
    s_ref[0] = jnp.dot(t, wg_ref[...], preferred_element_type=jnp.float32)


def _tc_mid_body(hvt_ref, agg_ref, bg_ref, w2_ref, b2_ref,
                 w1_ref, b1_ref, wg_ref, hvo_ref, s_ref):
    u = jnp.maximum(agg_ref[0] + bg_ref[...], 0.0)
    ut = lax.dot_general(w2_ref[...], u, _DN01,
                         preferred_element_type=jnp.float32) + b2_ref[...]
    hvt = hvt_ref[0] + ut
    hvo_ref[0] = hvt
    t = lax.dot_general(jnp.maximum(hvt, 0.0), w1_ref[...], _DN0,
                        preferred_element_type=jnp.float32) + b1_ref[...]
    s_ref[0] = jnp.dot(t, wg_ref[...], preferred_element_type=jnp.float32)


def _tc_last_body(hvt_ref, agg_ref, bg_ref, w2_ref, b2_ref, ws_ref, wo_ref,
                  hvo_ref, s_ref):
    u = jnp.maximum(agg_ref[0] + bg_ref[...], 0.0)
    ut = lax.dot_general(w2_ref[...], u, _DN01,
                         preferred_element_type=jnp.float32) + b2_ref[...]
    skip = lax.dot_general(ws_ref[...], hvt_ref[0], _DN0,
                           preferred_element_type=jnp.float32)
    hvt = skip + ut
    hvo_ref[0] = hvt
    s_ref[0] = lax.dot_general(hvt, wo_ref[...], _DN0,
                               preferred_element_type=jnp.float32)


def _blk_spec(c, w):
    return pl.BlockSpec((1, c, w), lambda b, i: (b, 0, i))


def _full_spec(shape):
    return pl.BlockSpec(shape, lambda b, i: tuple(0 for _ in shape))


def _tc_pre(xp, fp, vp, w1, b1, wg):
    h = w1.shape[1]
    return pl.pallas_call(
        _tc_pre_body,
        grid=(B, NVB),
        in_specs=[_blk_spec(61, VB), _blk_spec(64, VB), _blk_spec(3, VB),
                  _full_spec(w1.shape), _full_spec(b1.shape),
                  _full_spec(wg.shape)],
        out_specs=[_blk_spec(C, VB), _blk_spec(VB, h)],
        out_shape=[jax.ShapeDtypeStruct((B, C, VP), jnp.float32),
                   jax.ShapeDtypeStruct((B, VP, h), jnp.float32)],
    )(xp, fp, vp, w1, b1, wg)


def _tc_mid(hvt, agg, bg, w2, b2, w1, b1, wg):
    h = w1.shape[1]
    ha = agg.shape[2]
    return pl.pallas_call(
        _tc_mid_body,
        grid=(B, NVB),
        in_specs=[_blk_spec(C, VB), _blk_spec(VB, ha), _full_spec(bg.shape),
                  _full_spec(w2.shape), _full_spec(b2.shape),
                  _full_spec(w1.shape), _full_spec(b1.shape),
                  _full_spec(wg.shape)],
        out_specs=[_blk_spec(C, VB), _blk_spec(VB, h)],
        out_shape=[jax.ShapeDtypeStruct((B, C, VP), jnp.float32),
                   jax.ShapeDtypeStruct((B, VP, h), jnp.float32)],
    )(hvt, agg, bg, w2, b2, w1, b1, wg)


def _tc_last(hvt, agg, bg, w2, b2, ws, wo):
    co = ws.shape[1]
    so = wo.shape[1]
    ha = agg.shape[2]
    return pl.pallas_call(
        _tc_last_body,
        grid=(B, NVB),
        in_specs=[_blk_spec(C, VB), _blk_spec(VB, ha), _full_spec(bg.shape),
                  _full_spec(w2.shape), _full_spec(b2.shape),
                  _full_spec(ws.shape), _full_spec(wo.shape)],
        out_specs=[_blk_spec(co, VB), _blk_spec(VB, so)],
        out_shape=[jax.ShapeDtypeStruct((B, co, VP), jnp.float32),
                   jax.ShapeDtypeStruct((B, VP, so), jnp.float32)],
    )(hvt, agg, bg, w2, b2, ws, wo)


def kernel(feature_input, x, vertices, edge_index, edge_weight,
           W1s, b1s, Wgs, bgs, W2s, b2s,
           W1f, b1f, Wgf, bgf, W2f, b2f, Wsf, Wo, bo):
    src = edge_index[0]
    dst = edge_index[1]
    epad = EP - E
    srcp = jnp.concatenate([src, jnp.zeros((epad,), jnp.int32)])
    dstp = jnp.concatenate([dst, jnp.zeros((epad,), jnp.int32)])
    ewp = jnp.concatenate([edge_weight, jnp.zeros((epad,), jnp.float32)])
    srcp = srcp.reshape(16, NCHUNK, K)
    dstp = dstp.reshape(16, NCHUNK, K)
    ewp = ewp.reshape(16, NCHUNK, K)

    hvt, s = _tc_pre(x, feature_input, vertices,
                     W1s[0], b1s[0].reshape(1, -1), Wgs[0])
    for i in range(4):
        agg = _spmm64(s.reshape(ROWS, 64), srcp, dstp, ewp).reshape(B, VP, 64)
        if i < 3:
            hvt, s = _tc_mid(hvt, agg, bgs[i].reshape(1, -1),
                             W2s[i], b2s[i].reshape(-1, 1),
                             W1s[i + 1], b1s[i + 1].reshape(1, -1), Wgs[i + 1])
        else:
            hvt, s = _tc_mid(hvt, agg, bgs[3].reshape(1, -1),
                             W2s[3], b2s[3].reshape(-1, 1),
                             W1f, b1f.reshape(1, -1), Wgf)
    agg4 = _spmm32(s.reshape(ROWS, 32), srcp, dstp, ewp).reshape(B, VP, 32)
    wo_pad = jnp.pad(Wo, ((0, 0), (0, 16 - Wo.shape[1])))
    hvt5, s5 = _tc_last(hvt, agg4, bgf.reshape(1, -1), W2f,
                        b2f.reshape(-1, 1), Wsf, wo_pad)
    agg5 = _spmm16(s5.reshape(ROWS, 16), srcp, dstp, ewp)

    x_out = hvt5[:, :, :V]
    dv_v = agg5.reshape(B, VP, 16)[:, :V, :3] + bo
    dv = jnp.transpose(dv_v, (0, 2, 1))
    vertices_out = vertices + dv
    return (x_out, vertices_out, dv)


# speedup vs baseline: 1.0177x; 1.0075x over previous
import functools

import jax
import jax.numpy as jnp
from jax import lax
from jax.experimental import pallas as pl
from jax.experimental.pallas import tpu as pltpu
from jax.experimental.pallas import tpu_sc as plsc

V = 6890
E = 110240
B = 8
C = 128

VP = 6912
RPT = VP // 16
EPT = 6912
EP = 16 * EPT
K = 128
NCHUNK = EPT // K
BPC = B // 2


def _make_spmm(dpad):
    mesh = plsc.VectorSubcoreMesh(core_axis_name="c", subcore_axis_name="s")

    @functools.partial(
        pl.kernel,
        out_type=jax.ShapeDtypeStruct((B * VP, dpad), jnp.float32),
        mesh=mesh,
        compiler_params=pltpu.CompilerParams(use_tc_tiling_on_sc=False),
        scratch_types=[
            pltpu.VMEM((NCHUNK, K), jnp.int32),
            pltpu.VMEM((NCHUNK, K), jnp.int32),
            pltpu.VMEM((NCHUNK, K), jnp.int32),
            pltpu.VMEM((NCHUNK, K), jnp.float32),
            pltpu.VMEM((3, K, dpad), jnp.float32),
            pltpu.VMEM((3, K, dpad), jnp.float32),
            pltpu.VMEM((RPT // 2, dpad), jnp.float32),
            pltpu.VMEM_SHARED((VP, dpad), jnp.float32),
            pltpu.SemaphoreType.DMA,
            pltpu.SemaphoreType.DMA,
            pltpu.SemaphoreType.DMA,
            pltpu.SemaphoreType.DMA,
        ],
    )
    def spmm(support_hbm, src_hbm, dst_hbm, ew_hbm, out_hbm,
             src_r, src_a, dst_r, ew_r, rows_a, rows_b, zero_v, acc_sh,
             sem_a, sem_b, sem_sa, sem_sb):
        c = lax.axis_index("c")
        s = lax.axis_index("s")

        pltpu.sync_copy(src_hbm.at[s], src_r)
        pltpu.sync_copy(dst_hbm.at[s], dst_r)
        pltpu.sync_copy(ew_hbm.at[s], ew_r)

        zvec = jnp.zeros((16,), jnp.float32)

        def _zfill(i, carry):
            for j in range(dpad // 16):
                zero_v[i, pl.ds(j * 16, 16)] = zvec
            return carry
        lax.fori_loop(0, RPT // 2, _zfill, 0)

        def _scale3(base, buf):
            def _q(q, cc):
                n = base + q
                for g in range(K // 16):
                    wv = ew_r[n, pl.ds(g * 16, 16)]
                    for l in range(16):
                        w = wv[l]
                        e = g * 16 + l
                        for j in range(dpad // 16):
                            buf[q, e, pl.ds(j * 16, 16)] = (
                                buf[q, e, pl.ds(j * 16, 16)] * w)
                return cc
            lax.fori_loop(0, 3, _q, 0)

        NSUPER = NCHUNK // 6

        def _batch(bi, carry):
            b = c * BPC + bi
            roff = b * VP

            def _adj(n, cc):
                for j in range(K // 16):
                    src_a[n, pl.ds(j * 16, 16)] = (
                        src_r[n, pl.ds(j * 16, 16)] + roff)
                return cc
            lax.fori_loop(0, NCHUNK, _adj, 0)

            pltpu.sync_copy(zero_v, acc_sh.at[pl.ds(s * RPT, RPT // 2)])
            pltpu.sync_copy(zero_v,
                            acc_sh.at[pl.ds(s * RPT + RPT // 2, RPT // 2)])
            plsc.subcore_barrier()

            for q in range(3):
                pltpu.async_copy(
                    support_hbm.at[src_a.at[q]], rows_a.at[q], sem_a)
                pltpu.async_copy(
                    support_hbm.at[src_a.at[3 + q]], rows_b.at[q], sem_b)

            def _super(t, cc):
                base = 6 * t

                for q in range(3):
                    pltpu.make_async_copy(
                        support_hbm.at[src_a.at[base + q]],
                        rows_a.at[q], sem_a).wait()
                _scale3(base, rows_a)
                for q in range(3):
                    pltpu.async_copy(
                        rows_a.at[q], acc_sh.at[dst_r.at[base + q]], sem_sa,
                        add=True)

                for q in range(3):
                    pltpu.make_async_copy(
                        support_hbm.at[src_a.at[base + 3 + q]],
                        rows_b.at[q], sem_b).wait()
                _scale3(base + 3, rows_b)
                for q in range(3):
                    pltpu.async_copy(
                        rows_b.at[q], acc_sh.at[dst_r.at[base + 3 + q]], sem_sb,
                        add=True)

                for q in range(3):
                    pltpu.make_async_copy(
                        rows_a.at[q], acc_sh.at[dst_r.at[base + q]],
                        sem_sa).wait()

                @pl.when(t < NSUPER - 1)
                def _():
                    for q in range(3):
                        pltpu.async_copy(
                            support_hbm.at[src_a.at[base + 6 + q]],
                            rows_a.at[q], sem_a)

                for q in range(3):
                    pltpu.make_async_copy(
                        rows_b.at[q], acc_sh.at[dst_r.at[base + 3 + q]],
                        sem_sb).wait()

                @pl.when(t < NSUPER - 1)
                def _():
                    for q in range(3):
                        pltpu.async_copy(
                            support_hbm.at[src_a.at[base + 9 + q]],
                            rows_b.at[q], sem_b)
                return cc

            lax.fori_loop(0, NSUPER, _super, 0)
            plsc.subcore_barrier()

            pltpu.sync_copy(acc_sh.at[pl.ds(s * RPT, RPT)],
                            out_hbm.at[pl.ds(roff + s * RPT, RPT)])
            plsc.subcore_barrier()
            return carry

        lax.fori_loop(0, BPC, _batch, 0)

    return spmm


_spmm64 = _make_spmm(64)
_spmm32 = _make_spmm(32)
_spmm16 = _make_spmm(16)


ROWS = B * VP
RB = 1024
NRB = ROWS // RB


def _tc_pre_body(hv_ref, w1_ref, b1_ref, wg_ref, s_ref):
    t = jnp.maximum(hv_ref[...], 0.0)
    t = jnp.dot(t, w1_ref[...], preferred_element_type=jnp.float32) + b1_ref[...]
    s_ref[...] = jnp.dot(t, wg_ref[...], preferred_element_type=jnp.float32)


def _tc_mid_body(hv_ref, agg_ref, bg_ref, w2_ref, b2_ref,
                 w1_ref, b1_ref, wg_ref, hvo_ref, s_ref):
    u = jnp.maximum(agg_ref[...] + bg_ref[...], 0.0)
    u = jnp.dot(u, w2_ref[...], preferred_element_type=jnp.float32) + b2_ref[...]
    hv = hv_ref[...] + u
    hvo_ref[...] = hv
    t = jnp.maximum(hv, 0.0)
    t = jnp.dot(t, w1_ref[...], preferred_element_type=jnp.float32) + b1_ref[...]
    s_ref[...] = jnp.dot(t, wg_ref[...], preferred_element_type=jnp.float32)


def _tc_last_body(hv_ref, agg_ref, bg_ref, w2_ref, b2_ref, ws_ref, wo_ref,
                  hvo_ref, s_ref):
    u = jnp.maximum(agg_ref[...] + bg_ref[...], 0.0)
    u = jnp.dot(u, w2_ref[...], preferred_element_type=jnp.float32) + b2_ref[...]
    hv = jnp.dot(hv_ref[...], ws_ref[...], preferred_element_type=jnp.float32) + u
    hvo_ref[...] = hv
    s_ref[...] = jnp.dot(hv, wo_ref[...], preferred_element_type=jnp.float32)


def _row_spec(w):
    return pl.BlockSpec((RB, w), lambda i: (i, 0))


def _full_spec(shape):
    return pl.BlockSpec(shape, lambda i: tuple(0 for _ in shape))


def _tc_pre(hv, w1, b1, wg):
    h = w1.shape[1]
    return pl.pallas_call(
        _tc_pre_body,
        grid=(NRB,),
        in_specs=[_row_spec(C), _full_spec(w1.shape), _full_spec(b1.shape),
                  _full_spec(wg.shape)],
        out_specs=_row_spec(h),
        out_shape=jax.ShapeDtypeStruct((ROWS, h), jnp.float32),
    )(hv, w1, b1, wg)


def _tc_mid(hv, agg, bg, w2, b2, w1, b1, wg):
    h = w1.shape[1]
    return pl.pallas_call(
        _tc_mid_body,
        grid=(NRB,),
        in_specs=[_row_spec(C), _row_spec(agg.shape[1]), _full_spec(bg.shape),
                  _full_spec(w2.shape), _full_spec(b2.shape),
                  _full_spec(w1.shape), _full_spec(b1.shape),
                  _full_spec(wg.shape)],
        out_specs=[_row_spec(C), _row_spec(h)],
        out_shape=[jax.ShapeDtypeStruct((ROWS, C), jnp.float32),
                   jax.ShapeDtypeStruct((ROWS, h), jnp.float32)],
    )(hv, agg, bg, w2, b2, w1, b1, wg)


def _tc_last(hv, agg, bg, w2, b2, ws, wo):
    co = ws.shape[1]
    so = wo.shape[1]
    return pl.pallas_call(
        _tc_last_body,
        grid=(NRB,),
        in_specs=[_row_spec(C), _row_spec(agg.shape[1]), _full_spec(bg.shape),
                  _full_spec(w2.shape), _full_spec(b2.shape),
                  _full_spec(ws.shape), _full_spec(wo.shape)],
        out_specs=[_row_spec(co), _row_spec(so)],
        out_shape=[jax.ShapeDtypeStruct((ROWS, co), jnp.float32),
                   jax.ShapeDtypeStruct((ROWS, so), jnp.float32)],
    )(hv, agg, bg, w2, b2, ws, wo)


def kernel(feature_input, x, vertices, edge_index, edge_weight,
           W1s, b1s, Wgs, bgs, W2s, b2s,
           W1f, b1f, Wgf, bgf, W2f, b2f, Wsf, Wo, bo):
    src = edge_index[0]
    dst = edge_index[1]
    epad = EP - E
    srcp = jnp.concatenate([src, jnp.zeros((epad,), jnp.int32)])
    dstp = jnp.concatenate([dst, jnp.zeros((epad,), jnp.int32)])
    ewp = jnp.concatenate([edge_weight, jnp.zeros((epad,), jnp.float32)])
    srcp = srcp.reshape(16, NCHUNK, K)
    dstp = dstp.reshape(16, NCHUNK, K)
    ewp = ewp.reshape(16, NCHUNK, K)

    h = jnp.concatenate([x, feature_input, vertices], axis=1)
    hv = jnp.transpose(h, (0, 2, 1))
    hv = jnp.pad(hv, ((0, 0), (0, VP - V), (0, 0)))
    hv = hv.reshape(ROWS, C)

    s = _tc_pre(hv, W1s[0], b1s[0].reshape(1, -1), Wgs[0])
    for i in range(4):
        agg = _spmm64(s, srcp, dstp, ewp)
        if i < 3:
            hv, s = _tc_mid(hv, agg, bgs[i].reshape(1, -1),
                            W2s[i], b2s[i].reshape(1, -1),
                            W1s[i + 1], b1s[i + 1].reshape(1, -1), Wgs[i + 1])
        else:
            hv, s = _tc_mid(hv, agg, bgs[3].reshape(1, -1),
                            W2s[3], b2s[3].reshape(1, -1),
                            W1f, b1f.reshape(1, -1), Wgf)
    agg4 = _spmm32(s, srcp, dstp, ewp)
    wo_pad = jnp.pad(Wo, ((0, 0), (0, 16 - Wo.shape[1])))
    hv5, s5 = _tc_last(hv, agg4, bgf.reshape(1, -1), W2f, b2f.reshape(1, -1),
                       Wsf, wo_pad)
    agg5 = _spmm16(s5, srcp, dstp, ewp)

    x_out = jnp.transpose(hv5.reshape(B, VP, -1)[:, :V, :], (0, 2, 1))
    dv_v = agg5.reshape(B, VP, 16)[:, :V, :3] + bo
    dv = jnp.transpose(dv_v, (0, 2, 1))
    vertices_out = vertices + dv
    return (x_out, vertices_out, dv)


# speedup vs baseline: 1.0881x; 1.0692x over previous
import functools

import jax
import jax.numpy as jnp
from jax import lax
from jax.experimental import pallas as pl
from jax.experimental.pallas import tpu as pltpu
from jax.experimental.pallas import tpu_sc as plsc

V = 6890
E = 110240
B = 8
C = 128

VP = 6912
RPT = VP // 16
EPT = 6912
EP = 16 * EPT
K = 128
NCHUNK = EPT // K
BPC = B // 2


def _make_spmm(dpad):
    mesh = plsc.VectorSubcoreMesh(core_axis_name="c", subcore_axis_name="s")

    @functools.partial(
        pl.kernel,
        out_type=jax.ShapeDtypeStruct((B * VP, dpad), jnp.float32),
        mesh=mesh,
        compiler_params=pltpu.CompilerParams(use_tc_tiling_on_sc=False),
        scratch_types=[
            pltpu.VMEM((NCHUNK, K), jnp.int32),
            pltpu.VMEM((NCHUNK, K), jnp.int32),
            pltpu.VMEM((NCHUNK, K), jnp.int32),
            pltpu.VMEM((NCHUNK, K), jnp.float32),
            pltpu.VMEM((3, K, dpad), jnp.float32),
            pltpu.VMEM((3, K, dpad), jnp.float32),
            pltpu.VMEM((RPT // 2, dpad), jnp.float32),
            pltpu.VMEM_SHARED((VP, dpad), jnp.float32),
            pltpu.SemaphoreType.DMA,
            pltpu.SemaphoreType.DMA,
            pltpu.SemaphoreType.DMA,
            pltpu.SemaphoreType.DMA,
        ],
    )
    def spmm(support_hbm, src_hbm, dst_hbm, ew_hbm, out_hbm,
             src_r, src_a, dst_r, ew_r, rows_a, rows_b, zero_v, acc_sh,
             sem_a, sem_b, sem_sa, sem_sb):
        c = lax.axis_index("c")
        s = lax.axis_index("s")

        pltpu.sync_copy(src_hbm.at[s], src_r)
        pltpu.sync_copy(dst_hbm.at[s], dst_r)
        pltpu.sync_copy(ew_hbm.at[s], ew_r)

        zvec = jnp.zeros((16,), jnp.float32)

        def _zfill(i, carry):
            for j in range(dpad // 16):
                zero_v[i, pl.ds(j * 16, 16)] = zvec
            return carry
        lax.fori_loop(0, RPT // 2, _zfill, 0)

        def _scale3(base, buf):
            def _q(q, cc):
                n = base + q
                for g in range(K // 16):
                    wv = ew_r[n, pl.ds(g * 16, 16)]
                    for l in range(16):
                        w = wv[l]
                        e = g * 16 + l
                        for j in range(dpad // 16):
                            buf[q, e, pl.ds(j * 16, 16)] = (
                                buf[q, e, pl.ds(j * 16, 16)] * w)
                return cc
            lax.fori_loop(0, 3, _q, 0)

        NSUPER = NCHUNK // 6

        def _batch(bi, carry):
            b = c * BPC + bi
            roff = b * VP

            def _adj(n, cc):
                for j in range(K // 16):
                    src_a[n, pl.ds(j * 16, 16)] = (
                        src_r[n, pl.ds(j * 16, 16)] + roff)
                return cc
            lax.fori_loop(0, NCHUNK, _adj, 0)

            pltpu.sync_copy(zero_v, acc_sh.at[pl.ds(s * RPT, RPT // 2)])
            pltpu.sync_copy(zero_v,
                            acc_sh.at[pl.ds(s * RPT + RPT // 2, RPT // 2)])
            plsc.subcore_barrier()

            for q in range(3):
                pltpu.async_copy(
                    support_hbm.at[src_a.at[q]], rows_a.at[q], sem_a)
                pltpu.async_copy(
                    support_hbm.at[src_a.at[3 + q]], rows_b.at[q], sem_b)

            def _super(t, cc):
                base = 6 * t

                for q in range(3):
                    pltpu.make_async_copy(
                        support_hbm.at[src_a.at[base + q]],
                        rows_a.at[q], sem_a).wait()
                _scale3(base, rows_a)
                for q in range(3):
                    pltpu.async_copy(
                        rows_a.at[q], acc_sh.at[dst_r.at[base + q]], sem_sa,
                        add=True)

                for q in range(3):
                    pltpu.make_async_copy(
                        support_hbm.at[src_a.at[base + 3 + q]],
                        rows_b.at[q], sem_b).wait()
                _scale3(base + 3, rows_b)
                for q in range(3):
                    pltpu.async_copy(
                        rows_b.at[q], acc_sh.at[dst_r.at[base + 3 + q]], sem_sb,
                        add=True)

                for q in range(3):
                    pltpu.make_async_copy(
                        rows_a.at[q], acc_sh.at[dst_r.at[base + q]],
                        sem_sa).wait()

                @pl.when(t < NSUPER - 1)
                def _():
                    for q in range(3):
                        pltpu.async_copy(
                            support_hbm.at[src_a.at[base + 6 + q]],
                            rows_a.at[q], sem_a)

                for q in range(3):
                    pltpu.make_async_copy(
                        rows_b.at[q], acc_sh.at[dst_r.at[base + 3 + q]],
                        sem_sb).wait()

                @pl.when(t < NSUPER - 1)
                def _():
                    for q in range(3):
                        pltpu.async_copy(
                            support_hbm.at[src_a.at[base + 9 + q]],
                            rows_b.at[q], sem_b)
                return cc

            lax.fori_loop(0, NSUPER, _super, 0)
            plsc.subcore_barrier()

            pltpu.sync_copy(acc_sh.at[pl.ds(s * RPT, RPT)],
                            out_hbm.at[pl.ds(roff + s * RPT, RPT)])
            plsc.subcore_barrier()
            return carry

        lax.fori_loop(0, BPC, _batch, 0)

    return spmm


_spmm64 = _make_spmm(64)
_spmm32 = _make_spmm(32)
_spmm16 = _make_spmm(16)


ROWS = B * VP
RB = 2048
NRB = ROWS // RB


def _tc_pre_body(hv_ref, w1_ref, b1_ref, wg_ref, s_ref):
    t = jnp.maximum(hv_ref[...], 0.0)
    t = jnp.dot(t, w1_ref[...], preferred_element_type=jnp.float32) + b1_ref[...]
    s_ref[...] = jnp.dot(t, wg_ref[...], preferred_element_type=jnp.float32)


def _tc_mid_body(hv_ref, agg_ref, bg_ref, w2_ref, b2_ref,
                 w1_ref, b1_ref, wg_ref, hvo_ref, s_ref):
    u = jnp.maximum(agg_ref[...] + bg_ref[...], 0.0)
    u = jnp.dot(u, w2_ref[...], preferred_element_type=jnp.float32) + b2_ref[...]
    hv = hv_ref[...] + u
    hvo_ref[...] = hv
    t = jnp.maximum(hv, 0.0)
    t = jnp.dot(t, w1_ref[...], preferred_element_type=jnp.float32) + b1_ref[...]
    s_ref[...] = jnp.dot(t, wg_ref[...], preferred_element_type=jnp.float32)


def _tc_last_body(hv_ref, agg_ref, bg_ref, w2_ref, b2_ref, ws_ref, wo_ref,
                  hvo_ref, s_ref):
    u = jnp.maximum(agg_ref[...] + bg_ref[...], 0.0)
    u = jnp.dot(u, w2_ref[...], preferred_element_type=jnp.float32) + b2_ref[...]
    hv = jnp.dot(hv_ref[...], ws_ref[...], preferred_element_type=jnp.float32) + u
    hvo_ref[...] = hv
    s_ref[...] = jnp.dot(hv, wo_ref[...], preferred_element_type=jnp.float32)


def _row_spec(w):
    return pl.BlockSpec((RB, w), lambda i: (i, 0))


def _full_spec(shape):
    return pl.BlockSpec(shape, lambda i: tuple(0 for _ in shape))


def _tc_pre(hv, w1, b1, wg):
    h = w1.shape[1]
    return pl.pallas_call(
        _tc_pre_body,
        grid=(NRB,),
        in_specs=[_row_spec(C), _full_spec(w1.shape), _full_spec(b1.shape),
                  _full_spec(wg.shape)],
        out_specs=_row_spec(h),
        out_shape=jax.ShapeDtypeStruct((ROWS, h), jnp.float32),
    )(hv, w1, b1, wg)


def _tc_mid(hv, agg, bg, w2, b2, w1, b1, wg):
    h = w1.shape[1]
    return pl.pallas_call(
        _tc_mid_body,
        grid=(NRB,),
        in_specs=[_row_spec(C), _row_spec(agg.shape[1]), _full_spec(bg.shape),
                  _full_spec(w2.shape), _full_spec(b2.shape),
                  _full_spec(w1.shape), _full_spec(b1.shape),
                  _full_spec(wg.shape)],
        out_specs=[_row_spec(C), _row_spec(h)],
        out_shape=[jax.ShapeDtypeStruct((ROWS, C), jnp.float32),
                   jax.ShapeDtypeStruct((ROWS, h), jnp.float32)],
    )(hv, agg, bg, w2, b2, w1, b1, wg)


def _tc_last(hv, agg, bg, w2, b2, ws, wo):
    co = ws.shape[1]
    so = wo.shape[1]
    return pl.pallas_call(
        _tc_last_body,
        grid=(NRB,),
        in_specs=[_row_spec(C), _row_spec(agg.shape[1]), _full_spec(bg.shape),
                  _full_spec(w2.shape), _full_spec(b2.shape),
                  _full_spec(ws.shape), _full_spec(wo.shape)],
        out_specs=[_row_spec(co), _row_spec(so)],
        out_shape=[jax.ShapeDtypeStruct((ROWS, co), jnp.float32),
                   jax.ShapeDtypeStruct((ROWS, so), jnp.float32)],
    )(hv, agg, bg, w2, b2, ws, wo)


def kernel(feature_input, x, vertices, edge_index, edge_weight,
           W1s, b1s, Wgs, bgs, W2s, b2s,
           W1f, b1f, Wgf, bgf, W2f, b2f, Wsf, Wo, bo):
    src = edge_index[0]
    dst = edge_index[1]
    epad = EP - E
    srcp = jnp.concatenate([src, jnp.zeros((epad,), jnp.int32)])
    dstp = jnp.concatenate([dst, jnp.zeros((epad,), jnp.int32)])
    ewp = jnp.concatenate([edge_weight, jnp.zeros((epad,), jnp.float32)])
    srcp = srcp.reshape(16, NCHUNK, K)
    dstp = dstp.reshape(16, NCHUNK, K)
    ewp = ewp.reshape(16, NCHUNK, K)

    h = jnp.concatenate([x, feature_input, vertices], axis=1)
    hv = jnp.transpose(h, (0, 2, 1))
    hv = jnp.pad(hv, ((0, 0), (0, VP - V), (0, 0)))
    hv = hv.reshape(ROWS, C)

    s = _tc_pre(hv, W1s[0], b1s[0].reshape(1, -1), Wgs[0])
    for i in range(4):
        agg = _spmm64(s, srcp, dstp, ewp)
        if i < 3:
            hv, s = _tc_mid(hv, agg, bgs[i].reshape(1, -1),
                            W2s[i], b2s[i].reshape(1, -1),
                            W1s[i + 1], b1s[i + 1].reshape(1, -1), Wgs[i + 1])
        else:
            hv, s = _tc_mid(hv, agg, bgs[3].reshape(1, -1),
                            W2s[3], b2s[3].reshape(1, -1),
                            W1f, b1f.reshape(1, -1), Wgf)
    agg4 = _spmm32(s, srcp, dstp, ewp)
    wo_pad = jnp.pad(Wo, ((0, 0), (0, 16 - Wo.shape[1])))
    hv5, s5 = _tc_last(hv, agg4, bgf.reshape(1, -1), W2f, b2f.reshape(1, -1),
                       Wsf, wo_pad)
    agg5 = _spmm16(s5, srcp, dstp, ewp)

    x_out = jnp.transpose(hv5.reshape(B, VP, -1)[:, :V, :], (0, 2, 1))
    dv_v = agg5.reshape(B, VP, 16)[:, :V, :3] + bo
    dv = jnp.transpose(dv_v, (0, 2, 1))
    vertices_out = vertices + dv
    return (x_out, vertices_out, dv)


# speedup vs baseline: 1.1039x; 1.0145x over previous
import functools

import jax
import jax.numpy as jnp
from jax import lax
from jax.experimental import pallas as pl
from jax.experimental.pallas import tpu as pltpu
from jax.experimental.pallas import tpu_sc as plsc

V = 6890
E = 110240
B = 8
C = 128

VP = 6912
RPT = VP // 16
EPT = 6912
EP = 16 * EPT
K = 128
NCHUNK = EPT // K
BPC = B // 2


def _make_spmm(dpad):
    mesh = plsc.VectorSubcoreMesh(core_axis_name="c", subcore_axis_name="s")

    @functools.partial(
        pl.kernel,
        out_type=jax.ShapeDtypeStruct((B * VP, dpad), jnp.float32),
        mesh=mesh,
        compiler_params=pltpu.CompilerParams(use_tc_tiling_on_sc=False),
        scratch_types=[
            pltpu.VMEM((NCHUNK, K), jnp.int32),
            pltpu.VMEM((NCHUNK, K), jnp.int32),
            pltpu.VMEM((NCHUNK, K), jnp.int32),
            pltpu.VMEM((NCHUNK, K), jnp.float32),
            pltpu.VMEM((3, K, dpad), jnp.float32),
            pltpu.VMEM((3, K, dpad), jnp.float32),
            pltpu.VMEM((RPT // 2, dpad), jnp.float32),
            pltpu.VMEM_SHARED((VP, dpad), jnp.float32),
            pltpu.SemaphoreType.DMA,
            pltpu.SemaphoreType.DMA,
            pltpu.SemaphoreType.DMA,
            pltpu.SemaphoreType.DMA,
        ],
    )
    def spmm(support_hbm, src_hbm, dst_hbm, ew_hbm, out_hbm,
             src_r, src_a, dst_r, ew_r, rows_a, rows_b, zero_v, acc_sh,
             sem_a, sem_b, sem_sa, sem_sb):
        c = lax.axis_index("c")
        s = lax.axis_index("s")

        pltpu.sync_copy(src_hbm.at[s], src_r)
        pltpu.sync_copy(dst_hbm.at[s], dst_r)
        pltpu.sync_copy(ew_hbm.at[s], ew_r)

        zvec = jnp.zeros((16,), jnp.float32)

        def _zfill(i, carry):
            for j in range(dpad // 16):
                zero_v[i, pl.ds(j * 16, 16)] = zvec
            return carry
        lax.fori_loop(0, RPT // 2, _zfill, 0)

        def _scale3(base, buf):
            def _q(q, cc):
                n = base + q
                for g in range(K // 16):
                    wv = ew_r[n, pl.ds(g * 16, 16)]
                    for l in range(16):
                        w = wv[l]
                        e = g * 16 + l
                        for j in range(dpad // 16):
                            buf[q, e, pl.ds(j * 16, 16)] = (
                                buf[q, e, pl.ds(j * 16, 16)] * w)
                return cc
            lax.fori_loop(0, 3, _q, 0)

        NSUPER = NCHUNK // 6

        def _batch(bi, carry):
            b = c * BPC + bi
            roff = b * VP

            def _adj(n, cc):
                for j in range(K // 16):
                    src_a[n, pl.ds(j * 16, 16)] = (
                        src_r[n, pl.ds(j * 16, 16)] + roff)
                return cc
            lax.fori_loop(0, NCHUNK, _adj, 0)

            pltpu.sync_copy(zero_v, acc_sh.at[pl.ds(s * RPT, RPT // 2)])
            pltpu.sync_copy(zero_v,
                            acc_sh.at[pl.ds(s * RPT + RPT // 2, RPT // 2)])
            plsc.subcore_barrier()

            for q in range(3):
                pltpu.async_copy(
                    support_hbm.at[src_a.at[q]], rows_a.at[q], sem_a)
                pltpu.async_copy(
                    support_hbm.at[src_a.at[3 + q]], rows_b.at[q], sem_b)

            def _super(t, cc):
                base = 6 * t

                for q in range(3):
                    pltpu.make_async_copy(
                        support_hbm.at[src_a.at[base + q]],
                        rows_a.at[q], sem_a).wait()
                _scale3(base, rows_a)
                for q in range(3):
                    pltpu.async_copy(
                        rows_a.at[q], acc_sh.at[dst_r.at[base + q]], sem_sa,
                        add=True)

                for q in range(3):
                    pltpu.make_async_copy(
                        support_hbm.at[src_a.at[base + 3 + q]],
                        rows_b.at[q], sem_b).wait()
                _scale3(base + 3, rows_b)
                for q in range(3):
                    pltpu.async_copy(
                        rows_b.at[q], acc_sh.at[dst_r.at[base + 3 + q]], sem_sb,
                        add=True)

                for q in range(3):
                    pltpu.make_async_copy(
                        rows_a.at[q], acc_sh.at[dst_r.at[base + q]],
                        sem_sa).wait()

                @pl.when(t < NSUPER - 1)
                def _():
                    for q in range(3):
                        pltpu.async_copy(
                            support_hbm.at[src_a.at[base + 6 + q]],
                            rows_a.at[q], sem_a)

                for q in range(3):
                    pltpu.make_async_copy(
                        rows_b.at[q], acc_sh.at[dst_r.at[base + 3 + q]],
                        sem_sb).wait()

                @pl.when(t < NSUPER - 1)
                def _():
                    for q in range(3):
                        pltpu.async_copy(
                            support_hbm.at[src_a.at[base + 9 + q]],
                            rows_b.at[q], sem_b)
                return cc

            lax.fori_loop(0, NSUPER, _super, 0)
            plsc.subcore_barrier()

            pltpu.sync_copy(acc_sh.at[pl.ds(s * RPT, RPT)],
                            out_hbm.at[pl.ds(roff + s * RPT, RPT)])
            plsc.subcore_barrier()
            return carry

        lax.fori_loop(0, BPC, _batch, 0)

    return spmm


_spmm64 = _make_spmm(64)
_spmm32 = _make_spmm(32)
_spmm16 = _make_spmm(16)


ROWS = B * VP
RB = 3072
NRB = ROWS // RB


def _tc_pre_body(hv_ref, w1_ref, b1_ref, wg_ref, s_ref):
    t = jnp.maximum(hv_ref[...], 0.0)
    t = jnp.dot(t, w1_ref[...], preferred_element_type=jnp.float32) + b1_ref[...]
    s_ref[...] = jnp.dot(t, wg_ref[...], preferred_element_type=jnp.float32)


def _tc_mid_body(hv_ref, agg_ref, bg_ref, w2_ref, b2_ref,
                 w1_ref, b1_ref, wg_ref, hvo_ref, s_ref):
    u = jnp.maximum(agg_ref[...] + bg_ref[...], 0.0)
    u = jnp.dot(u, w2_ref[...], preferred_element_type=jnp.float32) + b2_ref[...]
    hv = hv_ref[...] + u
    hvo_ref[...] = hv
    t = jnp.maximum(hv, 0.0)
    t = jnp.dot(t, w1_ref[...], preferred_element_type=jnp.float32) + b1_ref[...]
    s_ref[...] = jnp.dot(t, wg_ref[...], preferred_element_type=jnp.float32)


def _tc_last_body(hv_ref, agg_ref, bg_ref, w2_ref, b2_ref, ws_ref, wo_ref,
                  hvo_ref, s_ref):
    u = jnp.maximum(agg_ref[...] + bg_ref[...], 0.0)
    u = jnp.dot(u, w2_ref[...], preferred_element_type=jnp.float32) + b2_ref[...]
    hv = jnp.dot(hv_ref[...], ws_ref[...], preferred_element_type=jnp.float32) + u
    hvo_ref[...] = hv
    s_ref[...] = jnp.dot(hv, wo_ref[...], preferred_element_type=jnp.float32)


def _row_spec(w):
    return pl.BlockSpec((RB, w), lambda i: (i, 0))


def _full_spec(shape):
    return pl.BlockSpec(shape, lambda i: tuple(0 for _ in shape))


def _tc_pre(hv, w1, b1, wg):
    h = w1.shape[1]
    return pl.pallas_call(
        _tc_pre_body,
        grid=(NRB,),
        in_specs=[_row_spec(C), _full_spec(w1.shape), _full_spec(b1.shape),
                  _full_spec(wg.shape)],
        out_specs=_row_spec(h),
        out_shape=jax.ShapeDtypeStruct((ROWS, h), jnp.float32),
    )(hv, w1, b1, wg)


def _tc_mid(hv, agg, bg, w2, b2, w1, b1, wg):
    h = w1.shape[1]
    return pl.pallas_call(
        _tc_mid_body,
        grid=(NRB,),
        in_specs=[_row_spec(C), _row_spec(agg.shape[1]), _full_spec(bg.shape),
                  _full_spec(w2.shape), _full_spec(b2.shape),
                  _full_spec(w1.shape), _full_spec(b1.shape),
                  _full_spec(wg.shape)],
        out_specs=[_row_spec(C), _row_spec(h)],
        out_shape=[jax.ShapeDtypeStruct((ROWS, C), jnp.float32),
                   jax.ShapeDtypeStruct((ROWS, h), jnp.float32)],
    )(hv, agg, bg, w2, b2, w1, b1, wg)


def _tc_last(hv, agg, bg, w2, b2, ws, wo):
    co = ws.shape[1]
    so = wo.shape[1]
    return pl.pallas_call(
        _tc_last_body,
        grid=(NRB,),
        in_specs=[_row_spec(C), _row_spec(agg.shape[1]), _full_spec(bg.shape),
                  _full_spec(w2.shape), _full_spec(b2.shape),
                  _full_spec(ws.shape), _full_spec(wo.shape)],
        out_specs=[_row_spec(co), _row_spec(so)],
        out_shape=[jax.ShapeDtypeStruct((ROWS, co), jnp.float32),
                   jax.ShapeDtypeStruct((ROWS, so), jnp.float32)],
    )(hv, agg, bg, w2, b2, ws, wo)


def kernel(feature_input, x, vertices, edge_index, edge_weight,
           W1s, b1s, Wgs, bgs, W2s, b2s,
           W1f, b1f, Wgf, bgf, W2f, b2f, Wsf, Wo, bo):
    src = edge_index[0]
    dst = edge_index[1]
    epad = EP - E
    srcp = jnp.concatenate([src, jnp.zeros((epad,), jnp.int32)])
    dstp = jnp.concatenate([dst, jnp.zeros((epad,), jnp.int32)])
    ewp = jnp.concatenate([edge_weight, jnp.zeros((epad,), jnp.float32)])
    srcp = srcp.reshape(16, NCHUNK, K)
    dstp = dstp.reshape(16, NCHUNK, K)
    ewp = ewp.reshape(16, NCHUNK, K)

    h = jnp.concatenate([x, feature_input, vertices], axis=1)
    hv = jnp.transpose(h, (0, 2, 1))
    hv = jnp.pad(hv, ((0, 0), (0, VP - V), (0, 0)))
    hv = hv.reshape(ROWS, C)

    s = _tc_pre(hv, W1s[0], b1s[0].reshape(1, -1), Wgs[0])
    for i in range(4):
        agg = _spmm64(s, srcp, dstp, ewp)
        if i < 3:
            hv, s = _tc_mid(hv, agg, bgs[i].reshape(1, -1),
                            W2s[i], b2s[i].reshape(1, -1),
                            W1s[i + 1], b1s[i + 1].reshape(1, -1), Wgs[i + 1])
        else:
            hv, s = _tc_mid(hv, agg, bgs[3].reshape(1, -1),
                            W2s[3], b2s[3].reshape(1, -1),
                            W1f, b1f.reshape(1, -1), Wgf)
    agg4 = _spmm32(s, srcp, dstp, ewp)
    wo_pad = jnp.pad(Wo, ((0, 0), (0, 16 - Wo.shape[1])))
    hv5, s5 = _tc_last(hv, agg4, bgf.reshape(1, -1), W2f, b2f.reshape(1, -1),
                       Wsf, wo_pad)
    agg5 = _spmm16(s5, srcp, dstp, ewp)

    x_out = jnp.transpose(hv5.reshape(B, VP, -1)[:, :V, :], (0, 2, 1))
    dv_v = agg5.reshape(B, VP, 16)[:, :V, :3] + bo
    dv = jnp.transpose(dv_v, (0, 2, 1))
    vertices_out = vertices + dv
    return (x_out, vertices_out, dv)
